# Initial kernel scaffold; baseline (speedup 1.0000x reference)
import functools

import jax
import jax.numpy as jnp
from jax import lax
from jax.experimental import pallas as pl
from jax.experimental.pallas import tpu as pltpu
from jax.experimental.pallas import tpu_sc as plsc

N = 10000
E = 320000
D = 128
G = 64
NC = 2
NS = 16
NW = NC * NS
EPW = E // NW
CH = 100
NCHK = EPW // CH
RPS = N // NS
NPAD = 10240
HPS = NPAD // NS
BLK = 2000
F32 = jnp.float32

_mesh = plsc.VectorSubcoreMesh(
    core_axis_name="c", subcore_axis_name="s", num_cores=NC, num_subcores=NS)


def _deg_body(dst_hbm, out_hbm, dbuf, hist, tmp, accb, hstage):
    cid = lax.axis_index("c")
    sid = lax.axis_index("s")
    w = cid * NS + sid

    def zero_hist(i, _):
        hist[pl.ds(i * 16, 16)] = jnp.zeros((16,), F32)
        return 0
    lax.fori_loop(0, NPAD // 16, zero_hist, 0)

    pltpu.sync_copy(dst_hbm.at[pl.ds(w * EPW, EPW)], dbuf)

    def count(e, _):
        i = dbuf[e]
        hist[i] = hist[i] + 1.0
        return 0
    lax.fori_loop(0, EPW, count, 0)

    pltpu.sync_copy(hist, hstage.at[sid])
    plsc.subcore_barrier()

    def zero_acc(i, _):
        accb[pl.ds(i * 16, 16)] = jnp.zeros((16,), F32)
        return 0
    lax.fori_loop(0, HPS // 16, zero_acc, 0)

    def red(i, _):
        pltpu.sync_copy(hstage.at[i, pl.ds(sid * HPS, HPS)], tmp)

        def add16(j, _):
            accb[pl.ds(j * 16, 16)] = accb[pl.ds(j * 16, 16)] + tmp[pl.ds(j * 16, 16)]
            return 0
        lax.fori_loop(0, HPS // 16, add16, 0)
        return 0
    lax.fori_loop(0, NS, red, 0)
    pltpu.sync_copy(accb, out_hbm.at[cid, pl.ds(sid * HPS, HPS)])


_sc_degree = pl.kernel(
    _deg_body,
    out_type=jax.ShapeDtypeStruct((NC, NPAD), F32),
    mesh=_mesh,
    scratch_types=[
        pltpu.VMEM((EPW,), jnp.int32),
        pltpu.VMEM((NPAD,), F32),
        pltpu.VMEM((HPS,), F32),
        pltpu.VMEM((HPS,), F32),
        pltpu.VMEM_SHARED((NS, NPAD), F32),
    ],
)


def _scat_body(g_hbm, src_hbm, dst_hbm, zeros_hbm, out_hbm,
               sidx, didx, rows, sem, acc):
    cid = lax.axis_index("c")
    sid = lax.axis_index("s")
    w = cid * NS + sid

    pltpu.sync_copy(zeros_hbm, acc.at[pl.ds(sid * RPS, RPS), :])
    pltpu.sync_copy(src_hbm.at[pl.ds(w * NCHK, NCHK), :], sidx)
    pltpu.sync_copy(dst_hbm.at[pl.ds(w * NCHK, NCHK), :], didx)
    plsc.subcore_barrier()

    def chunk(j, _):
        pltpu.async_copy(g_hbm.at[sidx.at[j]], rows, sem).wait()
        pltpu.sync_copy(rows, acc.at[didx.at[j]], add=True)
        return 0
    lax.fori_loop(0, NCHK, chunk, 0)

    plsc.subcore_barrier()
    pltpu.sync_copy(acc.at[pl.ds(sid * RPS, RPS), :],
                    out_hbm.at[cid, pl.ds(sid * RPS, RPS), :])


_sc_scatter = pl.kernel(
    _scat_body,
    out_type=jax.ShapeDtypeStruct((NC, N, D), F32),
    mesh=_mesh,
    scratch_types=[
        pltpu.VMEM((NCHK, CH), jnp.int32),
        pltpu.VMEM((NCHK, CH), jnp.int32),
        pltpu.VMEM((CH, D), F32),
        pltpu.SemaphoreType.DMA,
        pltpu.VMEM_SHARED((N, D), F32),
    ],
)


def _l1_body(x_ref, w_ref, degt_ref, dinv_ref, g_ref):
    deg = degt_ref[:, 0:1] + degt_ref[:, 1:2] + 1.0
    dinv = lax.rsqrt(deg)
    dinv_ref[...] = dinv
    h = lax.dot_general(x_ref[...], w_ref[...], (((1,), (0,)), ((), ())),
                        precision=lax.Precision.HIGHEST,
                        preferred_element_type=F32)
    g_ref[...] = dinv * h


def _tc_layer1(x, W1, degt):
    return pl.pallas_call(
        _l1_body,
        grid=(N // BLK,),
        in_specs=[
            pl.BlockSpec((BLK, D), lambda i: (i, 0)),
            pl.BlockSpec((D, D), lambda i: (0, 0)),
            pl.BlockSpec((BLK, 2), lambda i: (i, 0)),
        ],
        out_specs=[
            pl.BlockSpec((BLK, 1), lambda i: (i, 0)),
            pl.BlockSpec((BLK, D), lambda i: (i, 0)),
        ],
        out_shape=[
            jax.ShapeDtypeStruct((N, 1), F32),
            jax.ShapeDtypeStruct((N, D), F32),
        ],
    )(x, W1, degt)


def _lmid_body(acc_ref, g_ref, b_ref, dinv_ref, w_ref, out_ref):
    dinv = dinv_ref[...]
    node = dinv * (acc_ref[0] + acc_ref[1] + g_ref[...]) + b_ref[...]
    a = jnp.maximum(node, 0.0)
    h = lax.dot_general(a, w_ref[...], (((1,), (0,)), ((), ())),
                        precision=lax.Precision.HIGHEST,
                        preferred_element_type=F32)
    out_ref[...] = dinv * h


def _tc_layer(acc, g, b, dinv, W):
    return pl.pallas_call(
        _lmid_body,
        grid=(N // BLK,),
        in_specs=[
            pl.BlockSpec((NC, BLK, D), lambda i: (0, i, 0)),
            pl.BlockSpec((BLK, D), lambda i: (i, 0)),
            pl.BlockSpec((1, D), lambda i: (0, 0)),
            pl.BlockSpec((BLK, 1), lambda i: (i, 0)),
            pl.BlockSpec((D, D), lambda i: (0, 0)),
        ],
        out_specs=pl.BlockSpec((BLK, D), lambda i: (i, 0)),
        out_shape=jax.ShapeDtypeStruct((N, D), F32),
    )(acc, g, b.reshape(1, D), dinv, W)


def _pool_body(acc_ref, g_ref, b_ref, dinv_ref, bcol_ref, brow_ref,
               wc1_ref, bc1_ref, wc2_ref, bc2_ref, out_ref,
               ssum, scnt, smax):
    i = pl.program_id(0)
    h = dinv_ref[...] * (acc_ref[0] + acc_ref[1] + g_ref[...]) + b_ref[...]

    br = brow_ref[:, pl.ds(i * BLK, BLK)]
    seg = lax.broadcasted_iota(jnp.int32, (G, 1), 0)
    onehot = (br == seg).astype(F32)
    part_sum = lax.dot_general(onehot, h, (((1,), (0,)), ((), ())),
                               precision=lax.Precision.HIGHEST,
                               preferred_element_type=F32)
    part_cnt = jnp.sum(onehot, axis=1, keepdims=True)

    bcol = bcol_ref[...]

    def seg_max(gi, mx):
        m = (bcol == gi)
        v = jnp.max(jnp.where(m, h, -jnp.inf), axis=0, keepdims=True)
        return lax.dynamic_update_slice(mx, v, (gi, 0))
    part_max = lax.fori_loop(0, G, seg_max, jnp.full((G, D), -jnp.inf, F32))

    @pl.when(i == 0)
    def _():
        ssum[...] = part_sum
        scnt[...] = part_cnt
        smax[...] = part_max

    @pl.when(i > 0)
    def _():
        ssum[...] = ssum[...] + part_sum
        scnt[...] = scnt[...] + part_cnt
        smax[...] = jnp.maximum(smax[...], part_max)

    @pl.when(i == pl.num_programs(0) - 1)
    def _():
        mean = ssum[...] / jnp.maximum(scnt[...], 1.0)
        mx = smax[...]
        z = (lax.dot_general(mean, wc1_ref[0:D, :], (((1,), (0,)), ((), ())),
                             precision=lax.Precision.HIGHEST,
                             preferred_element_type=F32)
             + lax.dot_general(mx, wc1_ref[D:2 * D, :], (((1,), (0,)), ((), ())),
                               precision=lax.Precision.HIGHEST,
                               preferred_element_type=F32)
             + bc1_ref[...])
        z = jnp.maximum(z, 0.0)
        out_ref[...] = (lax.dot_general(z, wc2_ref[...], (((1,), (0,)), ((), ())),
                                        precision=lax.Precision.HIGHEST,
                                        preferred_element_type=F32)
                        + bc2_ref[...])


def _tc_pool(acc, g, b, dinv, bcol, brow, Wc1, bc1, Wc2, bc2):
    return pl.pallas_call(
        _pool_body,
        grid=(N // BLK,),
        in_specs=[
            pl.BlockSpec((NC, BLK, D), lambda i: (0, i, 0)),
            pl.BlockSpec((BLK, D), lambda i: (i, 0)),
            pl.BlockSpec((1, D), lambda i: (0, 0)),
            pl.BlockSpec((BLK, 1), lambda i: (i, 0)),
            pl.BlockSpec((BLK, 1), lambda i: (i, 0)),
            pl.BlockSpec((1, N), lambda i: (0, 0)),
            pl.BlockSpec((2 * D, D), lambda i: (0, 0)),
            pl.BlockSpec((1, D), lambda i: (0, 0)),
            pl.BlockSpec((D, 16), lambda i: (0, 0)),
            pl.BlockSpec((1, 16), lambda i: (0, 0)),
        ],
        out_specs=pl.BlockSpec((G, 16), lambda i: (0, 0)),
        out_shape=jax.ShapeDtypeStruct((G, 16), F32),
        scratch_shapes=[
            pltpu.VMEM((G, D), F32),
            pltpu.VMEM((G, 1), F32),
            pltpu.VMEM((G, D), F32),
        ],
    )(acc, g, b.reshape(1, D), dinv, bcol, brow,
      Wc1, bc1.reshape(1, D), Wc2, bc2.reshape(1, 16))


@jax.jit
def kernel(x, edge_index, batch, W1, b1, W2, b2, W3, b3, Wc1, bc1, Wc2, bc2):
    src = edge_index[0]
    dst = edge_index[1]
    src2d = src.reshape(NW * NCHK, CH)
    dst2d = dst.reshape(NW * NCHK, CH)
    zrows = jnp.zeros((RPS, D), F32)

    degp = _sc_degree(dst)
    degt = degp.T[:N]
    dinv, g1 = _tc_layer1(x, W1, degt)
    acc1 = _sc_scatter(g1, src2d, dst2d, zrows)
    g2 = _tc_layer(acc1, g1, b1, dinv, W2)
    acc2 = _sc_scatter(g2, src2d, dst2d, zrows)
    g3 = _tc_layer(acc2, g2, b2, dinv, W3)
    acc3 = _sc_scatter(g3, src2d, dst2d, zrows)

    bcol = batch.reshape(N, 1)
    brow = batch.reshape(1, N)
    return _tc_pool(acc3, g3, b3, dinv, bcol, brow, Wc1, bc1, Wc2, bc2)


# speedup vs baseline: 16.3787x; 16.3787x over previous
import functools

import jax
import jax.numpy as jnp
from jax import lax
from jax.experimental import pallas as pl
from jax.experimental.pallas import tpu as pltpu
from jax.experimental.pallas import tpu_sc as plsc

N = 10000
E = 320000
D = 128
G = 64
NC = 2
NS = 16
NW = NC * NS
EPW = E // NW
CH = 125
NCHK = EPW // CH
NSC = 10240
RPS = NSC // NS
BLK = 2000
F32 = jnp.float32

_mesh = plsc.VectorSubcoreMesh(
    core_axis_name="c", subcore_axis_name="s", num_cores=NC, num_subcores=NS)


DW = 16


def _deg_body(dst_hbm, ones_hbm, zdeg_hbm, out_hbm, didx, ones_v, acc):
    cid = lax.axis_index("c")
    sid = lax.axis_index("s")
    w = cid * NS + sid

    pltpu.sync_copy(zdeg_hbm, acc.at[pl.ds(sid * RPS, RPS), :])
    pltpu.sync_copy(ones_hbm, ones_v)
    pltpu.sync_copy(dst_hbm.at[pl.ds(w * NCHK, NCHK), :], didx)
    plsc.subcore_barrier()

    def chunk(j, _):
        pltpu.sync_copy(ones_v, acc.at[didx.at[j]], add=True)
        return 0
    lax.fori_loop(0, NCHK, chunk, 0)

    plsc.subcore_barrier()
    pltpu.sync_copy(acc.at[pl.ds(sid * RPS, RPS), :],
                    out_hbm.at[cid, pl.ds(sid * RPS, RPS), :])


_sc_degree = pl.kernel(
    _deg_body,
    out_type=jax.ShapeDtypeStruct((NC, NSC, DW), F32),
    mesh=_mesh,
    scratch_types=[
        pltpu.VMEM((NCHK, CH), jnp.int32),
        pltpu.VMEM((CH, DW), F32),
        pltpu.VMEM_SHARED((NSC, DW), F32),
    ],
)


def _scat_body(g_hbm, src_hbm, dst_hbm, zeros_hbm, out_hbm,
               sidx, didx, rows, sem, acc):
    cid = lax.axis_index("c")
    sid = lax.axis_index("s")
    w = cid * NS + sid

    pltpu.sync_copy(zeros_hbm, acc.at[pl.ds(sid * RPS, RPS), :])
    pltpu.sync_copy(src_hbm.at[pl.ds(w * NCHK, NCHK), :], sidx)
    pltpu.sync_copy(dst_hbm.at[pl.ds(w * NCHK, NCHK), :], didx)
    plsc.subcore_barrier()

    def chunk(j, _):
        pltpu.async_copy(g_hbm.at[sidx.at[j]], rows, sem).wait()
        pltpu.sync_copy(rows, acc.at[didx.at[j]], add=True)
        return 0
    lax.fori_loop(0, NCHK, chunk, 0)

    plsc.subcore_barrier()
    pltpu.sync_copy(acc.at[pl.ds(sid * RPS, RPS), :],
                    out_hbm.at[cid, pl.ds(sid * RPS, RPS), :])


_sc_scatter = pl.kernel(
    _scat_body,
    out_type=jax.ShapeDtypeStruct((NC, NSC, D), F32),
    mesh=_mesh,
    scratch_types=[
        pltpu.VMEM((NCHK, CH), jnp.int32),
        pltpu.VMEM((NCHK, CH), jnp.int32),
        pltpu.VMEM((CH, D), F32),
        pltpu.SemaphoreType.DMA,
        pltpu.VMEM_SHARED((NSC, D), F32),
    ],
)


def _l1_body(x_ref, w_ref, degt_ref, dinv_ref, g_ref):
    deg = degt_ref[0, :, 0:1] + degt_ref[1, :, 0:1] + 1.0
    dinv = lax.rsqrt(deg)
    dinv_ref[...] = dinv
    h = lax.dot_general(x_ref[...], w_ref[...], (((1,), (0,)), ((), ())),
                        precision=lax.Precision.HIGHEST,
                        preferred_element_type=F32)
    g_ref[...] = dinv * h


def _tc_layer1(x, W1, degt):
    return pl.pallas_call(
        _l1_body,
        grid=(N // BLK,),
        in_specs=[
            pl.BlockSpec((BLK, D), lambda i: (i, 0)),
            pl.BlockSpec((D, D), lambda i: (0, 0)),
            pl.BlockSpec((NC, BLK, DW), lambda i: (0, i, 0)),
        ],
        out_specs=[
            pl.BlockSpec((BLK, 1), lambda i: (i, 0)),
            pl.BlockSpec((BLK, D), lambda i: (i, 0)),
        ],
        out_shape=[
            jax.ShapeDtypeStruct((N, 1), F32),
            jax.ShapeDtypeStruct((N, D), F32),
        ],
    )(x, W1, degt)


def _lmid_body(acc_ref, g_ref, b_ref, dinv_ref, w_ref, out_ref):
    dinv = dinv_ref[...]
    node = dinv * (acc_ref[0] + acc_ref[1] + g_ref[...]) + b_ref[...]
    a = jnp.maximum(node, 0.0)
    h = lax.dot_general(a, w_ref[...], (((1,), (0,)), ((), ())),
                        precision=lax.Precision.HIGHEST,
                        preferred_element_type=F32)
    out_ref[...] = dinv * h


def _tc_layer(acc, g, b, dinv, W):
    return pl.pallas_call(
        _lmid_body,
        grid=(N // BLK,),
        in_specs=[
            pl.BlockSpec((NC, BLK, D), lambda i: (0, i, 0)),
            pl.BlockSpec((BLK, D), lambda i: (i, 0)),
            pl.BlockSpec((1, D), lambda i: (0, 0)),
            pl.BlockSpec((BLK, 1), lambda i: (i, 0)),
            pl.BlockSpec((D, D), lambda i: (0, 0)),
        ],
        out_specs=pl.BlockSpec((BLK, D), lambda i: (i, 0)),
        out_shape=jax.ShapeDtypeStruct((N, D), F32),
    )(acc, g, b.reshape(1, D), dinv, W)


def _pool_body(acc_ref, g_ref, b_ref, dinv_ref, bcol_ref,
               wc1_ref, bc1_ref, wc2_ref, bc2_ref, out_ref,
               ssum, scnt, smax):
    i = pl.program_id(0)
    h = dinv_ref[...] * (acc_ref[0] + acc_ref[1] + g_ref[...]) + b_ref[...]

    bcol = bcol_ref[...]
    seg = lax.broadcasted_iota(jnp.int32, (1, G), 1)
    onehot = (bcol == seg).astype(F32)
    part_sum = lax.dot_general(onehot, h, (((0,), (0,)), ((), ())),
                               precision=lax.Precision.HIGHEST,
                               preferred_element_type=F32)
    part_cnt = jnp.sum(onehot, axis=0)[:, None]

    @pl.when(i == 0)
    def _():
        ssum[...] = part_sum
        scnt[...] = part_cnt
        smax[...] = jnp.full((G, D), -jnp.inf, F32)

    @pl.when(i > 0)
    def _():
        ssum[...] = ssum[...] + part_sum
        scnt[...] = scnt[...] + part_cnt

    def seg_max(gi, _):
        m = (bcol == gi)
        v = jnp.max(jnp.where(m, h, -jnp.inf), axis=0, keepdims=True)
        smax[pl.ds(gi, 1), :] = jnp.maximum(smax[pl.ds(gi, 1), :], v)
        return 0
    lax.fori_loop(0, G, seg_max, 0)

    @pl.when(i == pl.num_programs(0) - 1)
    def _():
        mean = ssum[...] / jnp.maximum(scnt[...], 1.0)
        mx = smax[...]
        z = (lax.dot_general(mean, wc1_ref[0:D, :], (((1,), (0,)), ((), ())),
                             precision=lax.Precision.HIGHEST,
                             preferred_element_type=F32)
             + lax.dot_general(mx, wc1_ref[D:2 * D, :], (((1,), (0,)), ((), ())),
                               precision=lax.Precision.HIGHEST,
                               preferred_element_type=F32)
             + bc1_ref[...])
        z = jnp.maximum(z, 0.0)
        out_ref[...] = (lax.dot_general(z, wc2_ref[...], (((1,), (0,)), ((), ())),
                                        precision=lax.Precision.HIGHEST,
                                        preferred_element_type=F32)
                        + bc2_ref[...])


def _tc_pool(acc, g, b, dinv, bcol, Wc1, bc1, Wc2, bc2):
    return pl.pallas_call(
        _pool_body,
        grid=(N // BLK,),
        in_specs=[
            pl.BlockSpec((NC, BLK, D), lambda i: (0, i, 0)),
            pl.BlockSpec((BLK, D), lambda i: (i, 0)),
            pl.BlockSpec((1, D), lambda i: (0, 0)),
            pl.BlockSpec((BLK, 1), lambda i: (i, 0)),
            pl.BlockSpec((BLK, 1), lambda i: (i, 0)),
            pl.BlockSpec((2 * D, D), lambda i: (0, 0)),
            pl.BlockSpec((1, D), lambda i: (0, 0)),
            pl.BlockSpec((D, 16), lambda i: (0, 0)),
            pl.BlockSpec((1, 16), lambda i: (0, 0)),
        ],
        out_specs=pl.BlockSpec((G, 16), lambda i: (0, 0)),
        out_shape=jax.ShapeDtypeStruct((G, 16), F32),
        scratch_shapes=[
            pltpu.VMEM((G, D), F32),
            pltpu.VMEM((G, 1), F32),
            pltpu.VMEM((G, D), F32),
        ],
    )(acc, g, b.reshape(1, D), dinv, bcol,
      Wc1, bc1.reshape(1, D), Wc2, bc2.reshape(1, 16))


@jax.jit
def kernel(x, edge_index, batch, W1, b1, W2, b2, W3, b3, Wc1, bc1, Wc2, bc2):
    src = edge_index[0]
    dst = edge_index[1]
    src2d = src.reshape(NW * NCHK, CH)
    dst2d = dst.reshape(NW * NCHK, CH)
    zrows = jnp.zeros((RPS, D), F32)
    ones_rows = jnp.ones((CH, DW), F32)
    zdeg = jnp.zeros((RPS, DW), F32)

    degp = _sc_degree(dst2d, ones_rows, zdeg)
    dinv, g1 = _tc_layer1(x, W1, degp)
    acc1 = _sc_scatter(g1, src2d, dst2d, zrows)
    g2 = _tc_layer(acc1, g1, b1, dinv, W2)
    acc2 = _sc_scatter(g2, src2d, dst2d, zrows)
    g3 = _tc_layer(acc2, g2, b2, dinv, W3)
    acc3 = _sc_scatter(g3, src2d, dst2d, zrows)

    bcol = batch.reshape(N, 1)
    return _tc_pool(acc3, g3, b3, dinv, bcol, Wc1, bc1, Wc2, bc2)


# speedup vs baseline: 16.7289x; 1.0214x over previous
import functools

import jax
import jax.numpy as jnp
from jax import lax
from jax.experimental import pallas as pl
from jax.experimental.pallas import tpu as pltpu
from jax.experimental.pallas import tpu_sc as plsc

N = 10000
E = 320000
D = 128
G = 64
NC = 2
NS = 16
NW = NC * NS
EPW = E // NW
CH = 125
NCHK = EPW // CH
HCHK = NCHK // 2
HEPW = EPW // 2
NSC = 10240
RPS = NSC // NS
BLK = 2000
F32 = jnp.float32

_mesh = plsc.VectorSubcoreMesh(
    core_axis_name="c", subcore_axis_name="s", num_cores=NC, num_subcores=NS)


def _deg_body(dst_hbm, out_hbm, dflat, hist, tmp, tmp2, hstage):
    cid = lax.axis_index("c")
    sid = lax.axis_index("s")
    w = cid * NS + sid

    def zh(i, _):
        hist[pl.ds(i * 16, 16)] = jnp.zeros((16,), F32)
        return 0
    lax.fori_loop(0, NSC // 16, zh, 0)

    pltpu.sync_copy(dst_hbm.at[pl.ds(w * EPW, EPW)], dflat)
    ones16 = jnp.ones((16,), F32)

    def cnt(k, _):
        idx = dflat[pl.ds(k * 16, 16)]
        plsc.addupdate_scatter(hist, [idx], ones16)
        return 0
    lax.fori_loop(0, EPW // 16, cnt, 0)

    pltpu.sync_copy(hist, hstage.at[sid])
    plsc.subcore_barrier()

    def zt(i, _):
        tmp[pl.ds(i * 16, 16)] = jnp.zeros((16,), F32)
        return 0
    lax.fori_loop(0, RPS // 16, zt, 0)

    def red(i, _):
        pltpu.sync_copy(hstage.at[i, pl.ds(sid * RPS, RPS)], tmp2)

        def a16(k, _):
            tmp[pl.ds(k * 16, 16)] = tmp[pl.ds(k * 16, 16)] + tmp2[pl.ds(k * 16, 16)]
            return 0
        lax.fori_loop(0, RPS // 16, a16, 0)
        return 0
    lax.fori_loop(0, NS, red, 0)
    pltpu.sync_copy(tmp, out_hbm.at[cid, pl.ds(sid * RPS, RPS)])


_sc_degree = pl.kernel(
    _deg_body,
    out_type=jax.ShapeDtypeStruct((NC, NSC), F32),
    mesh=_mesh,
    compiler_params=pltpu.CompilerParams(needs_layout_passes=False),
    scratch_types=[
        pltpu.VMEM((EPW,), jnp.int32),
        pltpu.VMEM((NSC,), F32),
        pltpu.VMEM((RPS,), F32),
        pltpu.VMEM((RPS,), F32),
        pltpu.VMEM_SHARED((NS, NSC), F32),
    ],
)


def _scat_body(g_hbm, src_hbm, dst_hbm, zeros_hbm, out_hbm,
               sidx, didx, rows0, sem0, acc):
    cid = lax.axis_index("c")
    sid = lax.axis_index("s")
    w = cid * NS + sid

    pltpu.sync_copy(zeros_hbm, acc.at[pl.ds(sid * RPS, RPS), :])
    pltpu.sync_copy(src_hbm.at[pl.ds(w * NCHK, NCHK), :], sidx)
    pltpu.sync_copy(dst_hbm.at[pl.ds(w * NCHK, NCHK), :], didx)
    plsc.subcore_barrier()

    def chunk(j, _):
        pltpu.async_copy(g_hbm.at[sidx.at[j]], rows0, sem0).wait()
        pltpu.sync_copy(rows0, acc.at[didx.at[j]], add=True)
        return 0
    lax.fori_loop(0, NCHK, chunk, 0)

    plsc.subcore_barrier()
    pltpu.sync_copy(acc.at[pl.ds(sid * RPS, RPS), :],
                    out_hbm.at[cid, pl.ds(sid * RPS, RPS), :])


_sc_scatter = pl.kernel(
    _scat_body,
    out_type=jax.ShapeDtypeStruct((NC, NSC, D), F32),
    mesh=_mesh,
    scratch_types=[
        pltpu.VMEM((NCHK, CH), jnp.int32),
        pltpu.VMEM((NCHK, CH), jnp.int32),
        pltpu.VMEM((CH, D), F32),
        pltpu.SemaphoreType.DMA,
        pltpu.VMEM_SHARED((NSC, D), F32),
    ],
)


def _l1_body(x_ref, w_ref, degt_ref, dinv_ref, g_ref):
    deg = degt_ref[:, 0:1] + degt_ref[:, 1:2] + 1.0
    dinv = lax.rsqrt(deg)
    dinv_ref[...] = dinv
    h = lax.dot_general(x_ref[...], w_ref[...], (((1,), (0,)), ((), ())),
                        precision=lax.Precision.HIGHEST,
                        preferred_element_type=F32)
    g_ref[...] = dinv * h


def _tc_layer1(x, W1, degt):
    return pl.pallas_call(
        _l1_body,
        grid=(N // BLK,),
        in_specs=[
            pl.BlockSpec((BLK, D), lambda i: (i, 0)),
            pl.BlockSpec((D, D), lambda i: (0, 0)),
            pl.BlockSpec((BLK, 2), lambda i: (i, 0)),
        ],
        out_specs=[
            pl.BlockSpec((BLK, 1), lambda i: (i, 0)),
            pl.BlockSpec((BLK, D), lambda i: (i, 0)),
        ],
        out_shape=[
            jax.ShapeDtypeStruct((N, 1), F32),
            jax.ShapeDtypeStruct((N, D), F32),
        ],
    )(x, W1, degt)


def _lmid_body(acc_ref, g_ref, b_ref, dinv_ref, w_ref, out_ref):
    dinv = dinv_ref[...]
    node = dinv * (acc_ref[0] + acc_ref[1] + g_ref[...]) + b_ref[...]
    a = jnp.maximum(node, 0.0)
    h = lax.dot_general(a, w_ref[...], (((1,), (0,)), ((), ())),
                        precision=lax.Precision.HIGHEST,
                        preferred_element_type=F32)
    out_ref[...] = dinv * h


def _tc_layer(acc, g, b, dinv, W):
    return pl.pallas_call(
        _lmid_body,
        grid=(N // BLK,),
        in_specs=[
            pl.BlockSpec((NC, BLK, D), lambda i: (0, i, 0)),
            pl.BlockSpec((BLK, D), lambda i: (i, 0)),
            pl.BlockSpec((1, D), lambda i: (0, 0)),
            pl.BlockSpec((BLK, 1), lambda i: (i, 0)),
            pl.BlockSpec((D, D), lambda i: (0, 0)),
        ],
        out_specs=pl.BlockSpec((BLK, D), lambda i: (i, 0)),
        out_shape=jax.ShapeDtypeStruct((N, D), F32),
    )(acc, g, b.reshape(1, D), dinv, W)


def _pool_body(acc_ref, g_ref, b_ref, dinv_ref, bcol_ref,
               wc1_ref, bc1_ref, wc2_ref, bc2_ref, out_ref,
               ssum, scnt, smax):
    i = pl.program_id(0)
    h = dinv_ref[...] * (acc_ref[0] + acc_ref[1] + g_ref[...]) + b_ref[...]

    bcol = bcol_ref[...]
    seg = lax.broadcasted_iota(jnp.int32, (1, G), 1)
    onehot = (bcol == seg).astype(F32)
    part_sum = lax.dot_general(onehot, h, (((0,), (0,)), ((), ())),
                               precision=lax.Precision.HIGHEST,
                               preferred_element_type=F32)
    part_cnt = jnp.sum(onehot, axis=0)[:, None]

    @pl.when(i == 0)
    def _():
        ssum[...] = part_sum
        scnt[...] = part_cnt
        smax[...] = jnp.full((G, D), -jnp.inf, F32)

    @pl.when(i > 0)
    def _():
        ssum[...] = ssum[...] + part_sum
        scnt[...] = scnt[...] + part_cnt

    def seg_max(gi, _):
        m = (bcol == gi)
        v = jnp.max(jnp.where(m, h, -jnp.inf), axis=0, keepdims=True)
        smax[pl.ds(gi, 1), :] = jnp.maximum(smax[pl.ds(gi, 1), :], v)
        return 0
    lax.fori_loop(0, G, seg_max, 0)

    @pl.when(i == pl.num_programs(0) - 1)
    def _():
        mean = ssum[...] / jnp.maximum(scnt[...], 1.0)
        mx = smax[...]
        z = (lax.dot_general(mean, wc1_ref[0:D, :], (((1,), (0,)), ((), ())),
                             precision=lax.Precision.HIGHEST,
                             preferred_element_type=F32)
             + lax.dot_general(mx, wc1_ref[D:2 * D, :], (((1,), (0,)), ((), ())),
                               precision=lax.Precision.HIGHEST,
                               preferred_element_type=F32)
             + bc1_ref[...])
        z = jnp.maximum(z, 0.0)
        out_ref[...] = (lax.dot_general(z, wc2_ref[...], (((1,), (0,)), ((), ())),
                                        precision=lax.Precision.HIGHEST,
                                        preferred_element_type=F32)
                        + bc2_ref[...])


def _tc_pool(acc, g, b, dinv, bcol, Wc1, bc1, Wc2, bc2):
    return pl.pallas_call(
        _pool_body,
        grid=(N // BLK,),
        in_specs=[
            pl.BlockSpec((NC, BLK, D), lambda i: (0, i, 0)),
            pl.BlockSpec((BLK, D), lambda i: (i, 0)),
            pl.BlockSpec((1, D), lambda i: (0, 0)),
            pl.BlockSpec((BLK, 1), lambda i: (i, 0)),
            pl.BlockSpec((BLK, 1), lambda i: (i, 0)),
            pl.BlockSpec((2 * D, D), lambda i: (0, 0)),
            pl.BlockSpec((1, D), lambda i: (0, 0)),
            pl.BlockSpec((D, 16), lambda i: (0, 0)),
            pl.BlockSpec((1, 16), lambda i: (0, 0)),
        ],
        out_specs=pl.BlockSpec((G, 16), lambda i: (0, 0)),
        out_shape=jax.ShapeDtypeStruct((G, 16), F32),
        scratch_shapes=[
            pltpu.VMEM((G, D), F32),
            pltpu.VMEM((G, 1), F32),
            pltpu.VMEM((G, D), F32),
        ],
    )(acc, g, b.reshape(1, D), dinv, bcol,
      Wc1, bc1.reshape(1, D), Wc2, bc2.reshape(1, 16))


@jax.jit
def kernel(x, edge_index, batch, W1, b1, W2, b2, W3, b3, Wc1, bc1, Wc2, bc2):
    src = edge_index[0]
    dst = edge_index[1]
    src2d = src.reshape(NW * NCHK, CH)
    dst2d = dst.reshape(NW * NCHK, CH)
    zrows = jnp.zeros((RPS, D), F32)

    degp = _sc_degree(dst)
    degt = degp.T[:N]
    dinv, g1 = _tc_layer1(x, W1, degt)
    acc1 = _sc_scatter(g1, src2d, dst2d, zrows)
    g2 = _tc_layer(acc1, g1, b1, dinv, W2)
    acc2 = _sc_scatter(g2, src2d, dst2d, zrows)
    g3 = _tc_layer(acc2, g2, b2, dinv, W3)
    acc3 = _sc_scatter(g3, src2d, dst2d, zrows)

    bcol = batch.reshape(N, 1)
    return _tc_pool(acc3, g3, b3, dinv, bcol, Wc1, bc1, Wc2, bc2)


# speedup vs baseline: 18.6402x; 1.1142x over previous
import functools

import jax
import jax.numpy as jnp
from jax import lax
from jax.experimental import pallas as pl
from jax.experimental.pallas import tpu as pltpu
from jax.experimental.pallas import tpu_sc as plsc

N = 10000
E = 320000
D = 128
G = 64
NC = 2
NS = 16
NW = NC * NS
EPW = E // NW
CH = 125
NCHK = EPW // CH
HCHK = NCHK // 2
HEPW = EPW // 2
NSC = 10240
RPS = NSC // NS
BLK = 2000
F32 = jnp.float32

_mesh = plsc.VectorSubcoreMesh(
    core_axis_name="c", subcore_axis_name="s", num_cores=NC, num_subcores=NS)


def _deg_body(dst_hbm, out_hbm, dflat, hist, tmp, tmp2, hstage):
    cid = lax.axis_index("c")
    sid = lax.axis_index("s")
    w = cid * NS + sid

    def zh(i, _):
        hist[pl.ds(i * 16, 16)] = jnp.zeros((16,), F32)
        return 0
    lax.fori_loop(0, NSC // 16, zh, 0)

    pltpu.sync_copy(dst_hbm.at[pl.ds(w * EPW, EPW)], dflat)
    ones16 = jnp.ones((16,), F32)

    def cnt(k, _):
        idx = dflat[pl.ds(k * 16, 16)]
        plsc.addupdate_scatter(hist, [idx], ones16)
        return 0
    lax.fori_loop(0, EPW // 16, cnt, 0)

    pltpu.sync_copy(hist, hstage.at[sid])
    plsc.subcore_barrier()

    def zt(i, _):
        tmp[pl.ds(i * 16, 16)] = jnp.zeros((16,), F32)
        return 0
    lax.fori_loop(0, RPS // 16, zt, 0)

    def red(i, _):
        pltpu.sync_copy(hstage.at[i, pl.ds(sid * RPS, RPS)], tmp2)

        def a16(k, _):
            tmp[pl.ds(k * 16, 16)] = tmp[pl.ds(k * 16, 16)] + tmp2[pl.ds(k * 16, 16)]
            return 0
        lax.fori_loop(0, RPS // 16, a16, 0)
        return 0
    lax.fori_loop(0, NS, red, 0)
    pltpu.sync_copy(tmp, out_hbm.at[cid, pl.ds(sid * RPS, RPS)])


_sc_degree = pl.kernel(
    _deg_body,
    out_type=jax.ShapeDtypeStruct((NC, NSC), F32),
    mesh=_mesh,
    compiler_params=pltpu.CompilerParams(needs_layout_passes=False),
    scratch_types=[
        pltpu.VMEM((EPW,), jnp.int32),
        pltpu.VMEM((NSC,), F32),
        pltpu.VMEM((RPS,), F32),
        pltpu.VMEM((RPS,), F32),
        pltpu.VMEM_SHARED((NS, NSC), F32),
    ],
)


def _scat_body(g_hbm, src_hbm, dst_hbm, zeros_hbm, out_hbm,
               sidx, didx, rows0, rows1, sem0, sem1, acc):
    cid = lax.axis_index("c")
    sid = lax.axis_index("s")
    w = cid * NS + sid

    pltpu.sync_copy(zeros_hbm, acc.at[pl.ds(sid * RPS, RPS), :])
    plsc.subcore_barrier()

    slots = ((rows0, sem0), (rows1, sem1))
    for r in range(2):
        pltpu.sync_copy(src_hbm.at[pl.ds(w * NCHK + r * HCHK, HCHK), :], sidx)
        pltpu.sync_copy(dst_hbm.at[pl.ds(w * NCHK + r * HCHK, HCHK), :], didx)

        def outer(t, _):
            jj = t * 2
            descs = [pltpu.async_copy(g_hbm.at[sidx.at[jj + b]], rows, sem)
                     for b, (rows, sem) in enumerate(slots)]
            for b, (rows, sem) in enumerate(slots):
                descs[b].wait()
                pltpu.sync_copy(rows, acc.at[didx.at[jj + b]], add=True)
            return 0
        lax.fori_loop(0, HCHK // 2, outer, 0)

    plsc.subcore_barrier()
    pltpu.sync_copy(acc.at[pl.ds(sid * RPS, RPS), :],
                    out_hbm.at[cid, pl.ds(sid * RPS, RPS), :])


_sc_scatter = pl.kernel(
    _scat_body,
    out_type=jax.ShapeDtypeStruct((NC, NSC, D), F32),
    mesh=_mesh,
    scratch_types=[
        pltpu.VMEM((HCHK, CH), jnp.int32),
        pltpu.VMEM((HCHK, CH), jnp.int32),
        pltpu.VMEM((CH, D), F32),
        pltpu.VMEM((CH, D), F32),
        pltpu.SemaphoreType.DMA,
        pltpu.SemaphoreType.DMA,
        pltpu.VMEM_SHARED((NSC, D), F32),
    ],
)


def _l1_body(x_ref, w_ref, degt_ref, dinv_ref, g_ref):
    deg = degt_ref[:, 0:1] + degt_ref[:, 1:2] + 1.0
    dinv = lax.rsqrt(deg)
    dinv_ref[...] = dinv
    h = lax.dot_general(x_ref[...], w_ref[...], (((1,), (0,)), ((), ())),
                        precision=lax.Precision.HIGHEST,
                        preferred_element_type=F32)
    g_ref[...] = dinv * h


def _tc_layer1(x, W1, degt):
    return pl.pallas_call(
        _l1_body,
        grid=(N // BLK,),
        in_specs=[
            pl.BlockSpec((BLK, D), lambda i: (i, 0)),
            pl.BlockSpec((D, D), lambda i: (0, 0)),
            pl.BlockSpec((BLK, 2), lambda i: (i, 0)),
        ],
        out_specs=[
            pl.BlockSpec((BLK, 1), lambda i: (i, 0)),
            pl.BlockSpec((BLK, D), lambda i: (i, 0)),
        ],
        out_shape=[
            jax.ShapeDtypeStruct((N, 1), F32),
            jax.ShapeDtypeStruct((N, D), F32),
        ],
    )(x, W1, degt)


def _lmid_body(acc_ref, g_ref, b_ref, dinv_ref, w_ref, out_ref):
    dinv = dinv_ref[...]
    node = dinv * (acc_ref[0] + acc_ref[1] + g_ref[...]) + b_ref[...]
    a = jnp.maximum(node, 0.0)
    h = lax.dot_general(a, w_ref[...], (((1,), (0,)), ((), ())),
                        precision=lax.Precision.HIGHEST,
                        preferred_element_type=F32)
    out_ref[...] = dinv * h


def _tc_layer(acc, g, b, dinv, W):
    return pl.pallas_call(
        _lmid_body,
        grid=(N // BLK,),
        in_specs=[
            pl.BlockSpec((NC, BLK, D), lambda i: (0, i, 0)),
            pl.BlockSpec((BLK, D), lambda i: (i, 0)),
            pl.BlockSpec((1, D), lambda i: (0, 0)),
            pl.BlockSpec((BLK, 1), lambda i: (i, 0)),
            pl.BlockSpec((D, D), lambda i: (0, 0)),
        ],
        out_specs=pl.BlockSpec((BLK, D), lambda i: (i, 0)),
        out_shape=jax.ShapeDtypeStruct((N, D), F32),
    )(acc, g, b.reshape(1, D), dinv, W)


def _pool_body(acc_ref, g_ref, b_ref, dinv_ref, bcol_ref,
               wc1_ref, bc1_ref, wc2_ref, bc2_ref, out_ref,
               ssum, scnt, smax):
    i = pl.program_id(0)
    h = dinv_ref[...] * (acc_ref[0] + acc_ref[1] + g_ref[...]) + b_ref[...]

    bcol = bcol_ref[...]
    seg = lax.broadcasted_iota(jnp.int32, (1, G), 1)
    onehot = (bcol == seg).astype(F32)
    part_sum = lax.dot_general(onehot, h, (((0,), (0,)), ((), ())),
                               precision=lax.Precision.HIGHEST,
                               preferred_element_type=F32)
    part_cnt = jnp.sum(onehot, axis=0)[:, None]

    @pl.when(i == 0)
    def _():
        ssum[...] = part_sum
        scnt[...] = part_cnt
        smax[...] = jnp.full((G, D), -jnp.inf, F32)

    @pl.when(i > 0)
    def _():
        ssum[...] = ssum[...] + part_sum
        scnt[...] = scnt[...] + part_cnt

    def seg_max(gi, _):
        m = (bcol == gi)
        v = jnp.max(jnp.where(m, h, -jnp.inf), axis=0, keepdims=True)
        smax[pl.ds(gi, 1), :] = jnp.maximum(smax[pl.ds(gi, 1), :], v)
        return 0
    lax.fori_loop(0, G, seg_max, 0)

    @pl.when(i == pl.num_programs(0) - 1)
    def _():
        mean = ssum[...] / jnp.maximum(scnt[...], 1.0)
        mx = smax[...]
        z = (lax.dot_general(mean, wc1_ref[0:D, :], (((1,), (0,)), ((), ())),
                             precision=lax.Precision.HIGHEST,
                             preferred_element_type=F32)
             + lax.dot_general(mx, wc1_ref[D:2 * D, :], (((1,), (0,)), ((), ())),
                               precision=lax.Precision.HIGHEST,
                               preferred_element_type=F32)
             + bc1_ref[...])
        z = jnp.maximum(z, 0.0)
        out_ref[...] = (lax.dot_general(z, wc2_ref[...], (((1,), (0,)), ((), ())),
                                        precision=lax.Precision.HIGHEST,
                                        preferred_element_type=F32)
                        + bc2_ref[...])


def _tc_pool(acc, g, b, dinv, bcol, Wc1, bc1, Wc2, bc2):
    return pl.pallas_call(
        _pool_body,
        grid=(N // BLK,),
        in_specs=[
            pl.BlockSpec((NC, BLK, D), lambda i: (0, i, 0)),
            pl.BlockSpec((BLK, D), lambda i: (i, 0)),
            pl.BlockSpec((1, D), lambda i: (0, 0)),
            pl.BlockSpec((BLK, 1), lambda i: (i, 0)),
            pl.BlockSpec((BLK, 1), lambda i: (i, 0)),
            pl.BlockSpec((2 * D, D), lambda i: (0, 0)),
            pl.BlockSpec((1, D), lambda i: (0, 0)),
            pl.BlockSpec((D, 16), lambda i: (0, 0)),
            pl.BlockSpec((1, 16), lambda i: (0, 0)),
        ],
        out_specs=pl.BlockSpec((G, 16), lambda i: (0, 0)),
        out_shape=jax.ShapeDtypeStruct((G, 16), F32),
        scratch_shapes=[
            pltpu.VMEM((G, D), F32),
            pltpu.VMEM((G, 1), F32),
            pltpu.VMEM((G, D), F32),
        ],
    )(acc, g, b.reshape(1, D), dinv, bcol,
      Wc1, bc1.reshape(1, D), Wc2, bc2.reshape(1, 16))


@jax.jit
def kernel(x, edge_index, batch, W1, b1, W2, b2, W3, b3, Wc1, bc1, Wc2, bc2):
    src = edge_index[0]
    dst = edge_index[1]
    src2d = src.reshape(NW * NCHK, CH)
    dst2d = dst.reshape(NW * NCHK, CH)
    zrows = jnp.zeros((RPS, D), F32)

    degp = _sc_degree(dst)
    degt = degp.T[:N]
    dinv, g1 = _tc_layer1(x, W1, degt)
    acc1 = _sc_scatter(g1, src2d, dst2d, zrows)
    g2 = _tc_layer(acc1, g1, b1, dinv, W2)
    acc2 = _sc_scatter(g2, src2d, dst2d, zrows)
    g3 = _tc_layer(acc2, g2, b2, dinv, W3)
    acc3 = _sc_scatter(g3, src2d, dst2d, zrows)

    bcol = batch.reshape(N, 1)
    return _tc_pool(acc3, g3, b3, dinv, bcol, Wc1, bc1, Wc2, bc2)


# speedup vs baseline: 19.0526x; 1.0221x over previous
import functools

import jax
import jax.numpy as jnp
from jax import lax
from jax.experimental import pallas as pl
from jax.experimental.pallas import tpu as pltpu
from jax.experimental.pallas import tpu_sc as plsc

N = 10000
E = 320000
D = 128
G = 64
NC = 2
NS = 16
NW = NC * NS
EPW = E // NW
CH = 125
NCHK = EPW // CH
HCHK = NCHK // 2
HEPW = EPW // 2
NSC = 10240
RPS = NSC // NS
BLK = 2000
F32 = jnp.float32

_mesh = plsc.VectorSubcoreMesh(
    core_axis_name="c", subcore_axis_name="s", num_cores=NC, num_subcores=NS)


def _deg_body(dst_hbm, out_hbm, dflat, hist, tmp, tmp2, hstage):
    cid = lax.axis_index("c")
    sid = lax.axis_index("s")
    w = cid * NS + sid

    def zh(i, _):
        hist[pl.ds(i * 16, 16)] = jnp.zeros((16,), F32)
        return 0
    lax.fori_loop(0, NSC // 16, zh, 0)

    pltpu.sync_copy(dst_hbm.at[pl.ds(w * EPW, EPW)], dflat)
    ones16 = jnp.ones((16,), F32)

    def cnt(k, _):
        idx = dflat[pl.ds(k * 16, 16)]
        plsc.addupdate_scatter(hist, [idx], ones16)
        return 0
    lax.fori_loop(0, EPW // 16, cnt, 0)

    pltpu.sync_copy(hist, hstage.at[sid])
    plsc.subcore_barrier()

    def zt(i, _):
        tmp[pl.ds(i * 16, 16)] = jnp.zeros((16,), F32)
        return 0
    lax.fori_loop(0, RPS // 16, zt, 0)

    def red(i, _):
        pltpu.sync_copy(hstage.at[i, pl.ds(sid * RPS, RPS)], tmp2)

        def a16(k, _):
            tmp[pl.ds(k * 16, 16)] = tmp[pl.ds(k * 16, 16)] + tmp2[pl.ds(k * 16, 16)]
            return 0
        lax.fori_loop(0, RPS // 16, a16, 0)
        return 0
    lax.fori_loop(0, NS, red, 0)
    pltpu.sync_copy(tmp, out_hbm.at[cid, pl.ds(sid * RPS, RPS)])


_sc_degree = pl.kernel(
    _deg_body,
    out_type=jax.ShapeDtypeStruct((NC, NSC), F32),
    mesh=_mesh,
    compiler_params=pltpu.CompilerParams(needs_layout_passes=False),
    scratch_types=[
        pltpu.VMEM((EPW,), jnp.int32),
        pltpu.VMEM((NSC,), F32),
        pltpu.VMEM((RPS,), F32),
        pltpu.VMEM((RPS,), F32),
        pltpu.VMEM_SHARED((NS, NSC), F32),
    ],
)


def _scat_body(g_hbm, src_hbm, dst_hbm, zeros_hbm, out_hbm,
               sidx, didx, rows0, rows1, semg0, semg1, sems0, sems1, acc):
    cid = lax.axis_index("c")
    sid = lax.axis_index("s")
    w = cid * NS + sid

    pltpu.sync_copy(zeros_hbm, acc.at[pl.ds(sid * RPS, RPS), :])
    plsc.subcore_barrier()

    slots = ((rows0, semg0, sems0), (rows1, semg1, sems1))
    for r in range(2):
        pltpu.sync_copy(src_hbm.at[pl.ds(w * NCHK + r * HCHK, HCHK), :], sidx)
        pltpu.sync_copy(dst_hbm.at[pl.ds(w * NCHK + r * HCHK, HCHK), :], didx)
        pltpu.async_copy(g_hbm.at[sidx.at[0]], rows0, semg0)
        pltpu.async_copy(g_hbm.at[sidx.at[1]], rows1, semg1)

        def outer(t, _):
            jj = t * 2
            for b, (rows, semg, sems) in enumerate(slots):
                j = jj + b
                pltpu.make_async_copy(g_hbm.at[sidx.at[j]], rows, semg).wait()
                pltpu.async_copy(rows, acc.at[didx.at[j]], sems, add=True)
            for b, (rows, semg, sems) in enumerate(slots):
                j = jj + b

                @pl.when(j + 2 < HCHK)
                def _():
                    pltpu.make_async_copy(rows, acc.at[didx.at[j]], sems).wait()
                    pltpu.async_copy(g_hbm.at[sidx.at[j + 2]], rows, semg)
            return 0
        lax.fori_loop(0, HCHK // 2, outer, 0)

        for b, (rows, semg, sems) in enumerate(slots):
            pltpu.make_async_copy(rows, acc.at[didx.at[HCHK - 2 + b]], sems).wait()

    plsc.subcore_barrier()
    pltpu.sync_copy(acc.at[pl.ds(sid * RPS, RPS), :],
                    out_hbm.at[cid, pl.ds(sid * RPS, RPS), :])


_sc_scatter = pl.kernel(
    _scat_body,
    out_type=jax.ShapeDtypeStruct((NC, NSC, D), F32),
    mesh=_mesh,
    scratch_types=[
        pltpu.VMEM((HCHK, CH), jnp.int32),
        pltpu.VMEM((HCHK, CH), jnp.int32),
        pltpu.VMEM((CH, D), F32),
        pltpu.VMEM((CH, D), F32),
        pltpu.SemaphoreType.DMA,
        pltpu.SemaphoreType.DMA,
        pltpu.SemaphoreType.DMA,
        pltpu.SemaphoreType.DMA,
        pltpu.VMEM_SHARED((NSC, D), F32),
    ],
)


def _l1_body(x_ref, w_ref, degt_ref, dinv_ref, g_ref):
    deg = degt_ref[:, 0:1] + degt_ref[:, 1:2] + 1.0
    dinv = lax.rsqrt(deg)
    dinv_ref[...] = dinv
    h = lax.dot_general(x_ref[...], w_ref[...], (((1,), (0,)), ((), ())),
                        precision=lax.Precision.HIGHEST,
                        preferred_element_type=F32)
    g_ref[...] = dinv * h


def _tc_layer1(x, W1, degt):
    return pl.pallas_call(
        _l1_body,
        grid=(N // BLK,),
        in_specs=[
            pl.BlockSpec((BLK, D), lambda i: (i, 0)),
            pl.BlockSpec((D, D), lambda i: (0, 0)),
            pl.BlockSpec((BLK, 2), lambda i: (i, 0)),
        ],
        out_specs=[
            pl.BlockSpec((BLK, 1), lambda i: (i, 0)),
            pl.BlockSpec((BLK, D), lambda i: (i, 0)),
        ],
        out_shape=[
            jax.ShapeDtypeStruct((N, 1), F32),
            jax.ShapeDtypeStruct((N, D), F32),
        ],
    )(x, W1, degt)


def _lmid_body(acc_ref, g_ref, b_ref, dinv_ref, w_ref, out_ref):
    dinv = dinv_ref[...]
    node = dinv * (acc_ref[0] + acc_ref[1] + g_ref[...]) + b_ref[...]
    a = jnp.maximum(node, 0.0)
    h = lax.dot_general(a, w_ref[...], (((1,), (0,)), ((), ())),
                        precision=lax.Precision.HIGHEST,
                        preferred_element_type=F32)
    out_ref[...] = dinv * h


def _tc_layer(acc, g, b, dinv, W):
    return pl.pallas_call(
        _lmid_body,
        grid=(N // BLK,),
        in_specs=[
            pl.BlockSpec((NC, BLK, D), lambda i: (0, i, 0)),
            pl.BlockSpec((BLK, D), lambda i: (i, 0)),
            pl.BlockSpec((1, D), lambda i: (0, 0)),
            pl.BlockSpec((BLK, 1), lambda i: (i, 0)),
            pl.BlockSpec((D, D), lambda i: (0, 0)),
        ],
        out_specs=pl.BlockSpec((BLK, D), lambda i: (i, 0)),
        out_shape=jax.ShapeDtypeStruct((N, D), F32),
    )(acc, g, b.reshape(1, D), dinv, W)


def _pool_body(acc_ref, g_ref, b_ref, dinv_ref, bcol_ref,
               wc1_ref, bc1_ref, wc2_ref, bc2_ref, out_ref,
               ssum, scnt, smax):
    i = pl.program_id(0)
    h = dinv_ref[...] * (acc_ref[0] + acc_ref[1] + g_ref[...]) + b_ref[...]

    bcol = bcol_ref[...]
    seg = lax.broadcasted_iota(jnp.int32, (1, G), 1)
    onehot = (bcol == seg).astype(F32)
    part_sum = lax.dot_general(onehot, h, (((0,), (0,)), ((), ())),
                               precision=lax.Precision.HIGHEST,
                               preferred_element_type=F32)
    part_cnt = jnp.sum(onehot, axis=0)[:, None]

    @pl.when(i == 0)
    def _():
        ssum[...] = part_sum
        scnt[...] = part_cnt
        smax[...] = jnp.full((G, D), -jnp.inf, F32)

    @pl.when(i > 0)
    def _():
        ssum[...] = ssum[...] + part_sum
        scnt[...] = scnt[...] + part_cnt

    def seg_max(gi, _):
        m = (bcol == gi)
        v = jnp.max(jnp.where(m, h, -jnp.inf), axis=0, keepdims=True)
        smax[pl.ds(gi, 1), :] = jnp.maximum(smax[pl.ds(gi, 1), :], v)
        return 0
    lax.fori_loop(0, G, seg_max, 0)

    @pl.when(i == pl.num_programs(0) - 1)
    def _():
        mean = ssum[...] / jnp.maximum(scnt[...], 1.0)
        mx = smax[...]
        z = (lax.dot_general(mean, wc1_ref[0:D, :], (((1,), (0,)), ((), ())),
                             precision=lax.Precision.HIGHEST,
                             preferred_element_type=F32)
             + lax.dot_general(mx, wc1_ref[D:2 * D, :], (((1,), (0,)), ((), ())),
                               precision=lax.Precision.HIGHEST,
                               preferred_element_type=F32)
             + bc1_ref[...])
        z = jnp.maximum(z, 0.0)
        out_ref[...] = (lax.dot_general(z, wc2_ref[...], (((1,), (0,)), ((), ())),
                                        precision=lax.Precision.HIGHEST,
                                        preferred_element_type=F32)
                        + bc2_ref[...])


def _tc_pool(acc, g, b, dinv, bcol, Wc1, bc1, Wc2, bc2):
    return pl.pallas_call(
        _pool_body,
        grid=(N // BLK,),
        in_specs=[
            pl.BlockSpec((NC, BLK, D), lambda i: (0, i, 0)),
            pl.BlockSpec((BLK, D), lambda i: (i, 0)),
            pl.BlockSpec((1, D), lambda i: (0, 0)),
            pl.BlockSpec((BLK, 1), lambda i: (i, 0)),
            pl.BlockSpec((BLK, 1), lambda i: (i, 0)),
            pl.BlockSpec((2 * D, D), lambda i: (0, 0)),
            pl.BlockSpec((1, D), lambda i: (0, 0)),
            pl.BlockSpec((D, 16), lambda i: (0, 0)),
            pl.BlockSpec((1, 16), lambda i: (0, 0)),
        ],
        out_specs=pl.BlockSpec((G, 16), lambda i: (0, 0)),
        out_shape=jax.ShapeDtypeStruct((G, 16), F32),
        scratch_shapes=[
            pltpu.VMEM((G, D), F32),
            pltpu.VMEM((G, 1), F32),
            pltpu.VMEM((G, D), F32),
        ],
    )(acc, g, b.reshape(1, D), dinv, bcol,
      Wc1, bc1.reshape(1, D), Wc2, bc2.reshape(1, 16))


@jax.jit
def kernel(x, edge_index, batch, W1, b1, W2, b2, W3, b3, Wc1, bc1, Wc2, bc2):
    src = edge_index[0]
    dst = edge_index[1]
    src2d = src.reshape(NW * NCHK, CH)
    dst2d = dst.reshape(NW * NCHK, CH)
    zrows = jnp.zeros((RPS, D), F32)

    degp = _sc_degree(dst)
    degt = degp.T[:N]
    dinv, g1 = _tc_layer1(x, W1, degt)
    acc1 = _sc_scatter(g1, src2d, dst2d, zrows)
    g2 = _tc_layer(acc1, g1, b1, dinv, W2)
    acc2 = _sc_scatter(g2, src2d, dst2d, zrows)
    g3 = _tc_layer(acc2, g2, b2, dinv, W3)
    acc3 = _sc_scatter(g3, src2d, dst2d, zrows)

    bcol = batch.reshape(N, 1)
    return _tc_pool(acc3, g3, b3, dinv, bcol, Wc1, bc1, Wc2, bc2)
